# baseline (speedup 1.0000x reference)
import functools

import jax
import jax.numpy as jnp
from jax import lax
from jax.experimental import pallas as pl
from jax.experimental.pallas import tpu as pltpu
from jax.experimental.pallas import tpu_sc as plsc

N = 10000
NP = 10240
E = 160000
D = 300

NC = 2
NS = 16
NW = NC * NS
K = 125
NA = (E // NS) // K
NB = (E // NW) // K
SEG = 40
SH = NP // NS
G = 128
EPD = NP * NS
DS = EPD // NW // 16

_mesh = plsc.VectorSubcoreMesh(core_axis_name="c", subcore_axis_name="s")


def _deg_body(cols16_hbm, zeros_np_hbm, deg_hbm, colsv, hist):
    c = lax.axis_index("c")
    s = lax.axis_index("s")
    w = c * NS + s
    pltpu.sync_copy(zeros_np_hbm, hist)
    pltpu.sync_copy(cols16_hbm.at[pl.ds(w * DS, DS)], colsv)
    ones16 = jnp.ones((16,), jnp.float32)

    def step(j, carry):
        plsc.addupdate_scatter(hist, [colsv[j]], ones16)
        return carry

    lax.fori_loop(0, DS, step, 0)
    pltpu.sync_copy(hist, deg_hbm.at[pl.ds(w * NP, NP)])


_deg_call = functools.partial(
    pl.kernel,
    mesh=_mesh,
    compiler_params=pltpu.CompilerParams(needs_layout_passes=False),
    out_type=jax.ShapeDtypeStruct((NW * NP,), jnp.float32),
    scratch_types=[
        pltpu.VMEM((DS, 16), jnp.int32),
        pltpu.VMEM((NP,), jnp.float32),
    ],
)(_deg_body)


def _mm1_body(x_ref, w_ref, y_ref):
    y_ref[...] = lax.dot_general(
        x_ref[...], w_ref[...], (((1,), (1,)), ((), ())),
        preferred_element_type=jnp.float32,
    )


def _mm1_call(x, W, bm=2048):
    return pl.pallas_call(
        _mm1_body,
        grid=(NP // bm,),
        in_specs=[
            pl.BlockSpec((bm, D), lambda i: (i, 0)),
            pl.BlockSpec((D, D), lambda i: (0, 0)),
        ],
        out_specs=pl.BlockSpec((bm, D), lambda i: (i, 0)),
        out_shape=jax.ShapeDtypeStruct((NP, D), jnp.float32),
    )(x, W)


def _mm2_body(y_ref, degp_ref, z_ref, dis_ref):
    ones_w = jnp.ones((NW, 1), jnp.float32)
    deg = lax.dot_general(
        degp_ref[...], ones_w, (((0,), (0,)), ((), ())),
        preferred_element_type=jnp.float32,
    ) + 1.0
    dis = lax.rsqrt(deg)
    z = y_ref[...] * dis
    bm = z.shape[0]
    z_ref[0] = z[:, :G]
    z_ref[1] = z[:, G:2 * G]
    z_ref[2] = jnp.concatenate(
        [z[:, 2 * G:], jnp.zeros((bm, 3 * G - D), jnp.float32)], axis=1)
    dis_ref[...] = jnp.broadcast_to(dis, (bm, 16))


def _mm2_call(y, degp, bm=2048):
    return pl.pallas_call(
        _mm2_body,
        grid=(NP // bm,),
        in_specs=[
            pl.BlockSpec((bm, D), lambda i: (i, 0)),
            pl.BlockSpec((NW, bm), lambda i: (0, i)),
        ],
        out_specs=[
            pl.BlockSpec((3, bm, G), lambda i: (0, i, 0)),
            pl.BlockSpec((bm, 16), lambda i: (i, 0)),
        ],
        out_shape=[
            jax.ShapeDtypeStruct((3, NP, G), jnp.float32),
            jax.ShapeDtypeStruct((NP, 16), jnp.float32),
        ],
    )(y, degp)


def _agg_body(rows_hbm, cols_hbm, z_hbm, zeros_hbm, s_hbm,
              idxr, idxc, buf0, buf1, sem0, sem1, acc):
    c = lax.axis_index("c")
    s = lax.axis_index("s")
    w = c * NS + s
    pltpu.sync_copy(zeros_hbm, acc.at[pl.ds(s * SH, SH)])

    def run_seg(g):
        zt = z_hbm.at[g]
        pltpu.async_copy(zt.at[idxr.at[0]], buf0, sem0)

        def pair(j0):
            pltpu.async_copy(zt.at[idxr.at[j0 + 1]], buf1, sem1)
            pltpu.make_async_copy(zt.at[idxr.at[j0]], buf0, sem0).wait()
            pltpu.sync_copy(buf0, acc.at[idxc.at[j0]], add=True)

            @pl.when(j0 + 2 < SEG)
            def _():
                pltpu.async_copy(zt.at[idxr.at[j0 + 2]], buf0, sem0)

            pltpu.make_async_copy(zt.at[idxr.at[j0 + 1]], buf1, sem1).wait()
            pltpu.sync_copy(buf1, acc.at[idxc.at[j0 + 1]], add=True)

        pl.loop(0, SEG, step=2)(pair)

    for t in range(NA // SEG):
        pltpu.sync_copy(
            rows_hbm.at[pl.ds(s * NA + t * SEG, SEG)], idxr)
        pltpu.sync_copy(cols_hbm.at[pl.ds(s * NA + t * SEG, SEG)], idxc)
        if t == 0:
            plsc.subcore_barrier()
        run_seg(c)
    plsc.subcore_barrier()
    pltpu.sync_copy(acc.at[pl.ds(s * SH, SH)],
                    s_hbm.at[pl.ds(c * NP + s * SH, SH)])
    pltpu.sync_copy(zeros_hbm, acc.at[pl.ds(s * SH, SH)])
    pltpu.sync_copy(rows_hbm.at[pl.ds(w * NB, NB)], idxr)
    pltpu.sync_copy(cols_hbm.at[pl.ds(w * NB, NB)], idxc)
    plsc.subcore_barrier()
    run_seg(2)
    plsc.subcore_barrier()
    pltpu.sync_copy(acc.at[pl.ds(s * SH, SH)],
                    s_hbm.at[pl.ds((2 + c) * NP + s * SH, SH)])


_agg_call = functools.partial(
    pl.kernel,
    mesh=_mesh,
    compiler_params=pltpu.CompilerParams(needs_layout_passes=False),
    out_type=jax.ShapeDtypeStruct((4 * NP, G), jnp.float32),
    scratch_types=[
        pltpu.VMEM((SEG, K), jnp.int32),
        pltpu.VMEM((SEG, K), jnp.int32),
        pltpu.VMEM((K, G), jnp.float32),
        pltpu.VMEM((K, G), jnp.float32),
        pltpu.SemaphoreType.DMA,
        pltpu.SemaphoreType.DMA,
        pltpu.VMEM_SHARED((NP, G), jnp.float32),
    ],
)(_agg_body)


def _epi_body(s_ref, z_ref, dis_ref, b_ref, out_ref):
    g0 = s_ref[0] + z_ref[0]
    g1 = s_ref[1] + z_ref[1]
    g2 = s_ref[2] + s_ref[3] + z_ref[2]
    h = jnp.concatenate([g0, g1, g2[:, :D - 2 * G]], axis=1)
    h = h * dis_ref[:, :1] + b_ref[...]
    h = jnp.maximum(h, 0.0)
    nrm = jnp.sqrt(jnp.sum(h * h, axis=1, keepdims=True))
    out_ref[...] = h / jnp.maximum(nrm, 1e-12)


def _epi_call(s4, z3, dis, b2, bm=2000):
    return pl.pallas_call(
        _epi_body,
        grid=(N // bm,),
        in_specs=[
            pl.BlockSpec((4, bm, G), lambda i: (0, i, 0)),
            pl.BlockSpec((3, bm, G), lambda i: (0, i, 0)),
            pl.BlockSpec((bm, 16), lambda i: (i, 0)),
            pl.BlockSpec((1, D), lambda i: (0, 0)),
        ],
        out_specs=pl.BlockSpec((bm, D), lambda i: (i, 0)),
        out_shape=jax.ShapeDtypeStruct((N, D), jnp.float32),
    )(s4, z3, dis, b2)


def kernel(x, adj, W, b):
    rows = adj[0].reshape(NS * NA, K)
    cols = adj[1].reshape(NS * NA, K)
    cols16 = jnp.concatenate(
        [adj[1], jnp.full((EPD - E,), N, jnp.int32)]).reshape(NP * NS // 16, 16)
    zeros_np = jnp.zeros((NP,), jnp.float32)
    zeros_sh = jnp.zeros((SH, G), jnp.float32)

    y = _mm1_call(x, W)
    deg1d = _deg_call(cols16, zeros_np)
    z3, dis = _mm2_call(y, deg1d.reshape(NW, NP))
    s4 = _agg_call(rows, cols, z3, zeros_sh)
    out = _epi_call(s4.reshape(4, NP, G), z3, dis, b.reshape(1, D))
    return out


# speedup vs baseline: 1.0475x; 1.0475x over previous
import functools

import jax
import jax.numpy as jnp
from jax import lax
from jax.experimental import pallas as pl
from jax.experimental.pallas import tpu as pltpu
from jax.experimental.pallas import tpu_sc as plsc

N = 10000
NP = 10240
E = 160000
D = 300

NC = 2
NS = 16
NW = NC * NS
K = 125
NA = (E // NS) // K
NB = (E // NW) // K
SEG = 40
SH = NP // NS
G = 128
EPD = NP * NS
DS = EPD // NW // 16

_mesh = plsc.VectorSubcoreMesh(core_axis_name="c", subcore_axis_name="s")


def _deg_body(cols16_hbm, zeros_np_hbm, deg_hbm, colsv, hist):
    c = lax.axis_index("c")
    s = lax.axis_index("s")
    w = c * NS + s
    pltpu.sync_copy(zeros_np_hbm, hist)
    pltpu.sync_copy(cols16_hbm.at[pl.ds(w * DS, DS)], colsv)
    ones16 = jnp.ones((16,), jnp.float32)

    def step(j, carry):
        plsc.addupdate_scatter(hist, [colsv[j]], ones16)
        return carry

    lax.fori_loop(0, DS, step, 0)
    pltpu.sync_copy(hist, deg_hbm.at[pl.ds(w * NP, NP)])


_deg_call = functools.partial(
    pl.kernel,
    mesh=_mesh,
    compiler_params=pltpu.CompilerParams(needs_layout_passes=False),
    out_type=jax.ShapeDtypeStruct((NW * NP,), jnp.float32),
    scratch_types=[
        pltpu.VMEM((DS, 16), jnp.int32),
        pltpu.VMEM((NP,), jnp.float32),
    ],
)(_deg_body)


def _mm_body(x_ref, w_ref, degp_ref, z_ref, dis_ref):
    ones_w = jnp.ones((NW, 1), jnp.float32)
    deg = lax.dot_general(
        degp_ref[...], ones_w, (((0,), (0,)), ((), ())),
        preferred_element_type=jnp.float32,
    ) + 1.0
    dis = lax.rsqrt(deg)
    y = lax.dot_general(
        x_ref[...], w_ref[...], (((1,), (1,)), ((), ())),
        preferred_element_type=jnp.float32,
    )
    z = y * dis
    bm = z.shape[0]
    z_ref[0] = z[:, :G]
    z_ref[1] = z[:, G:2 * G]
    z_ref[2] = jnp.concatenate(
        [z[:, 2 * G:], jnp.zeros((bm, 3 * G - D), jnp.float32)], axis=1)
    dis_ref[...] = jnp.broadcast_to(dis, (bm, 16))


def _mm_call(x, W, degp, bm=2048):
    return pl.pallas_call(
        _mm_body,
        grid=(NP // bm,),
        in_specs=[
            pl.BlockSpec((bm, D), lambda i: (i, 0)),
            pl.BlockSpec((D, D), lambda i: (0, 0)),
            pl.BlockSpec((NW, bm), lambda i: (0, i)),
        ],
        out_specs=[
            pl.BlockSpec((3, bm, G), lambda i: (0, i, 0)),
            pl.BlockSpec((bm, 16), lambda i: (i, 0)),
        ],
        out_shape=[
            jax.ShapeDtypeStruct((3, NP, G), jnp.float32),
            jax.ShapeDtypeStruct((NP, 16), jnp.float32),
        ],
    )(x, W, degp)


def _agg_body(rows_hbm, cols_hbm, z_hbm, zeros_hbm, s_hbm,
              idxr, idxc, buf0, buf1, sem0, sem1, acc):
    c = lax.axis_index("c")
    s = lax.axis_index("s")
    w = c * NS + s
    pltpu.sync_copy(zeros_hbm, acc.at[pl.ds(s * SH, SH)])

    def run_seg(g):
        zt = z_hbm.at[g]
        pltpu.async_copy(zt.at[idxr.at[0]], buf0, sem0)

        def pair(j0):
            pltpu.async_copy(zt.at[idxr.at[j0 + 1]], buf1, sem1)
            pltpu.make_async_copy(zt.at[idxr.at[j0]], buf0, sem0).wait()
            pltpu.sync_copy(buf0, acc.at[idxc.at[j0]], add=True)

            @pl.when(j0 + 2 < SEG)
            def _():
                pltpu.async_copy(zt.at[idxr.at[j0 + 2]], buf0, sem0)

            pltpu.make_async_copy(zt.at[idxr.at[j0 + 1]], buf1, sem1).wait()
            pltpu.sync_copy(buf1, acc.at[idxc.at[j0 + 1]], add=True)

        pl.loop(0, SEG, step=2)(pair)

    for t in range(NA // SEG):
        pltpu.sync_copy(
            rows_hbm.at[pl.ds(s * NA + t * SEG, SEG)], idxr)
        pltpu.sync_copy(cols_hbm.at[pl.ds(s * NA + t * SEG, SEG)], idxc)
        if t == 0:
            plsc.subcore_barrier()
        run_seg(c)
    plsc.subcore_barrier()
    pltpu.sync_copy(acc.at[pl.ds(s * SH, SH)],
                    s_hbm.at[pl.ds(c * NP + s * SH, SH)])
    pltpu.sync_copy(zeros_hbm, acc.at[pl.ds(s * SH, SH)])
    pltpu.sync_copy(rows_hbm.at[pl.ds(w * NB, NB)], idxr)
    pltpu.sync_copy(cols_hbm.at[pl.ds(w * NB, NB)], idxc)
    plsc.subcore_barrier()
    run_seg(2)
    plsc.subcore_barrier()
    pltpu.sync_copy(acc.at[pl.ds(s * SH, SH)],
                    s_hbm.at[pl.ds((2 + c) * NP + s * SH, SH)])


_agg_call = functools.partial(
    pl.kernel,
    mesh=_mesh,
    compiler_params=pltpu.CompilerParams(needs_layout_passes=False),
    out_type=jax.ShapeDtypeStruct((4 * NP, G), jnp.float32),
    scratch_types=[
        pltpu.VMEM((SEG, K), jnp.int32),
        pltpu.VMEM((SEG, K), jnp.int32),
        pltpu.VMEM((K, G), jnp.float32),
        pltpu.VMEM((K, G), jnp.float32),
        pltpu.SemaphoreType.DMA,
        pltpu.SemaphoreType.DMA,
        pltpu.VMEM_SHARED((NP, G), jnp.float32),
    ],
)(_agg_body)


def _epi_body(s_ref, z_ref, dis_ref, b_ref, out_ref):
    g0 = s_ref[0] + z_ref[0]
    g1 = s_ref[1] + z_ref[1]
    g2 = s_ref[2] + s_ref[3] + z_ref[2]
    h = jnp.concatenate([g0, g1, g2[:, :D - 2 * G]], axis=1)
    h = h * dis_ref[:, :1] + b_ref[...]
    h = jnp.maximum(h, 0.0)
    nrm = jnp.sqrt(jnp.sum(h * h, axis=1, keepdims=True))
    out_ref[...] = h / jnp.maximum(nrm, 1e-12)


def _epi_call(s4, z3, dis, b2, bm=2000):
    return pl.pallas_call(
        _epi_body,
        grid=(N // bm,),
        in_specs=[
            pl.BlockSpec((4, bm, G), lambda i: (0, i, 0)),
            pl.BlockSpec((3, bm, G), lambda i: (0, i, 0)),
            pl.BlockSpec((bm, 16), lambda i: (i, 0)),
            pl.BlockSpec((1, D), lambda i: (0, 0)),
        ],
        out_specs=pl.BlockSpec((bm, D), lambda i: (i, 0)),
        out_shape=jax.ShapeDtypeStruct((N, D), jnp.float32),
    )(s4, z3, dis, b2)


def kernel(x, adj, W, b):
    rows = adj[0].reshape(NS * NA, K)
    cols = adj[1].reshape(NS * NA, K)
    cols16 = jnp.concatenate(
        [adj[1], jnp.full((EPD - E,), N, jnp.int32)]).reshape(NP * NS // 16, 16)
    zeros_np = jnp.zeros((NP,), jnp.float32)
    zeros_sh = jnp.zeros((SH, G), jnp.float32)

    deg1d = _deg_call(cols16, zeros_np)
    z3, dis = _mm_call(x, W, deg1d.reshape(NW, NP))
    s4 = _agg_call(rows, cols, z3, zeros_sh)
    out = _epi_call(s4.reshape(4, NP, G), z3, dis, b.reshape(1, D))
    return out
